# Initial kernel scaffold; baseline (speedup 1.0000x reference)
import jax
import jax.numpy as jnp
from jax.experimental import pallas as pl


def kernel(pillar_features, coords):
    raise NotImplementedError("write your pallas kernel here")


# speedup vs baseline: 7.0023x; 7.0023x over previous
import functools

import jax
import jax.numpy as jnp
from jax import lax
from jax.experimental import pallas as pl
from jax.experimental.pallas import tpu as pltpu
from jax.experimental.pallas import tpu_sc as plsc

NX = 496
NY = 496
C = 64
B = 4
P = 25000

L = 16
NCELL = NX * NY
SLICE = NCELL // 16
P_PAD = 25600
T = 25008
W = 3968
N_WIN = NCELL // W
SENT = 1 << 28

_mesh = plsc.VectorSubcoreMesh(core_axis_name="c", subcore_axis_name="s")


@functools.partial(
    pl.kernel,
    out_type=jax.ShapeDtypeStruct((B, C, NCELL), jnp.float32),
    mesh=_mesh,
    compiler_params=pltpu.CompilerParams(needs_layout_passes=False),
    scratch_types=[
        pltpu.VMEM((SLICE,), jnp.int32),
        pltpu.VMEM((P_PAD,), jnp.int32),
        pltpu.VMEM((2 * T,), jnp.float32),
        pltpu.VMEM((W,), jnp.int32),
        pltpu.VMEM((2, W), jnp.float32),
        pltpu.VMEM_SHARED((NCELL,), jnp.int32),
    ],
)
def _scatter_kernel(cells_hbm, feat_hbm, out_hbm,
                    map_v, cells_v, tab_v, idx_v, outw_v, map_sh):
    core = lax.axis_index("c")
    sub = lax.axis_index("s")
    iota = jnp.arange(L, dtype=jnp.int32)
    zeros_i = jnp.zeros((L,), jnp.int32)

    base = sub * SLICE

    for lb in range(2):
        b = 2 * core + lb

        pltpu.sync_copy(cells_hbm.at[b], cells_v)

        def _zero(i, _):
            map_v[pl.ds(i * L, L)] = zeros_i
            return 0
        lax.fori_loop(0, SLICE // L, _zero, 0)

        def _scan(k, _):
            cell = cells_v[pl.ds(k * L, L)]
            rel = cell - base
            ok = (rel >= 0) & (rel < SLICE)
            loc = jnp.clip(rel, 0, SLICE - 1)
            vals = k * L + 1 + iota
            plsc.store_scatter(map_v, [loc], vals, mask=ok)
            for _fix in range(2):
                rb = plsc.load_gather(map_v, [loc], mask=ok)
                redo = ok & (rb < vals)
                plsc.store_scatter(map_v, [loc], vals, mask=redo)
            return 0
        lax.fori_loop(0, P_PAD // L, _scan, 0)

        pltpu.sync_copy(map_v, map_sh.at[pl.ds(base, SLICE)])
        plsc.subcore_barrier()

        c_base = 4 * sub
        for p in range(2):
            c0 = c_base + 2 * p
            pltpu.sync_copy(feat_hbm.at[pl.ds((b * C + c0) * T, 2 * T)], tab_v)

            def _window(w, _):
                pltpu.sync_copy(map_sh.at[pl.ds(w * W, W)], idx_v)

                def _chunk(i, _):
                    idx = idx_v[pl.ds(i * L, L)]
                    outw_v[0, pl.ds(i * L, L)] = plsc.load_gather(tab_v, [idx])
                    outw_v[1, pl.ds(i * L, L)] = plsc.load_gather(tab_v, [idx + T])
                    return 0
                lax.fori_loop(0, W // L, _chunk, 0)

                pltpu.sync_copy(outw_v, out_hbm.at[b, pl.ds(c0, 2), pl.ds(w * W, W)])
                return 0
            lax.fori_loop(0, N_WIN, _window, 0)

        plsc.subcore_barrier()


def kernel(pillar_features, coords):
    x = coords[:, :, 1]
    y = coords[:, :, 2]
    keep = (x + y) > 0
    cells = jnp.where(keep, x * NY + y, SENT).astype(jnp.int32)
    cells = jnp.pad(cells, ((0, 0), (0, P_PAD - P)), constant_values=SENT)

    feat_t = jnp.transpose(pillar_features, (0, 2, 1))
    feat_t = jnp.pad(feat_t, ((0, 0), (0, 0), (1, T - P - 1)))
    feat_flat = feat_t.reshape(B * C * T)

    out = _scatter_kernel(cells, feat_flat)
    return out.reshape(B, C, NX, NY)


# speedup vs baseline: 7.7041x; 1.1002x over previous
import functools

import jax
import jax.numpy as jnp
from jax import lax
from jax.experimental import pallas as pl
from jax.experimental.pallas import tpu as pltpu
from jax.experimental.pallas import tpu_sc as plsc

NX = 496
NY = 496
C = 64
B = 4
P = 25000

L = 16
NCELL = NX * NY
SLICE = NCELL // 16
P_PAD = 25600
T = 25008
W = 3968
N_WIN = NCELL // W
SENT = 1 << 28

_mesh = plsc.VectorSubcoreMesh(core_axis_name="c", subcore_axis_name="s")


@functools.partial(
    pl.kernel,
    out_type=jax.ShapeDtypeStruct((B, C, NCELL), jnp.float32),
    mesh=_mesh,
    compiler_params=pltpu.CompilerParams(needs_layout_passes=False),
    scratch_types=[
        pltpu.VMEM((SLICE,), jnp.int32),
        pltpu.VMEM((P_PAD,), jnp.int32),
        pltpu.VMEM((2 * T,), jnp.float32),
        pltpu.VMEM((2 * W,), jnp.int32),
        pltpu.VMEM((4, W), jnp.float32),
        pltpu.VMEM_SHARED((NCELL,), jnp.int32),
        pltpu.SemaphoreType.DMA,
        pltpu.SemaphoreType.DMA,
        pltpu.SemaphoreType.DMA,
        pltpu.SemaphoreType.DMA,
    ],
)
def _scatter_kernel(cells_hbm, feat_hbm, out_hbm,
                    map_v, cells_v, tab_v, idx_v, outw_v, map_sh,
                    sem_i0, sem_i1, sem_o0, sem_o1):
    sem_i = (sem_i0, sem_i1)
    sem_o = (sem_o0, sem_o1)
    core = lax.axis_index("c")
    sub = lax.axis_index("s")
    iota = jnp.arange(L, dtype=jnp.int32)
    zeros_i = jnp.zeros((L,), jnp.int32)

    base = sub * SLICE

    for lb in range(2):
        b = 2 * core + lb

        pltpu.sync_copy(cells_hbm.at[b], cells_v)

        def _zero(i, _):
            map_v[pl.ds(i * L, L)] = zeros_i
            return 0
        lax.fori_loop(0, SLICE // L, _zero, 0)

        def _scan(k, _):
            cell = cells_v[pl.ds(k * L, L)]
            rel = cell - base
            ok = (rel >= 0) & (rel < SLICE)
            loc = jnp.clip(rel, 0, SLICE - 1)
            vals = k * L + 1 + iota
            plsc.store_scatter(map_v, [loc], vals, mask=ok)
            for _fix in range(2):
                rb = plsc.load_gather(map_v, [loc], mask=ok)
                redo = ok & (rb < vals)
                plsc.store_scatter(map_v, [loc], vals, mask=redo)
            return 0
        lax.fori_loop(0, P_PAD // L, _scan, 0)

        pltpu.sync_copy(map_v, map_sh.at[pl.ds(base, SLICE)])
        plsc.subcore_barrier()

        c_base = 4 * sub
        for p in range(2):
            c0 = c_base + 2 * p
            pltpu.sync_copy(feat_hbm.at[pl.ds((b * C + c0) * T, 2 * T)], tab_v)

            def idx_copy(w, j):
                return pltpu.make_async_copy(
                    map_sh.at[pl.ds(w * W, W)],
                    idx_v.at[pl.ds(j * W, W)], sem_i[j])

            def out_copy(w, j):
                return pltpu.make_async_copy(
                    outw_v.at[pl.ds(2 * j, 2)],
                    out_hbm.at[b, pl.ds(c0, 2), pl.ds(w * W, W)], sem_o[j])

            def gather_win(j):
                r0 = 2 * j

                def _chunk(i, _):
                    for u in range(4):
                        o = i * 4 * L + u * L
                        idx = idx_v[pl.ds(j * W + o, L)]
                        outw_v[r0, pl.ds(o, L)] = plsc.load_gather(tab_v, [idx])
                        outw_v[r0 + 1, pl.ds(o, L)] = plsc.load_gather(
                            tab_v, [idx + T])
                    return 0
                lax.fori_loop(0, W // (4 * L), _chunk, 0)

            idx_copy(0, 0).start()
            idx_copy(1, 1).start()
            for j in range(2):
                idx_copy(j, j).wait()
                gather_win(j)
                out_copy(j, j).start()
                idx_copy(j + 2, j).start()

            def _dbl(k, _):
                for j in range(2):
                    w = 2 * k + j
                    idx_copy(w, j).wait()
                    out_copy(w - 2, j).wait()
                    gather_win(j)
                    out_copy(w, j).start()
                    idx_copy(jnp.minimum(w + 2, N_WIN - 1), j).start()
                return 0
            lax.fori_loop(1, N_WIN // 2, _dbl, 0)

            for j in range(2):
                idx_copy(N_WIN - 2 + j, j).wait()
                out_copy(N_WIN - 2 + j, j).wait()

        plsc.subcore_barrier()


def kernel(pillar_features, coords):
    x = coords[:, :, 1]
    y = coords[:, :, 2]
    keep = (x + y) > 0
    cells = jnp.where(keep, x * NY + y, SENT).astype(jnp.int32)
    cells = jnp.pad(cells, ((0, 0), (0, P_PAD - P)), constant_values=SENT)

    feat_t = jnp.transpose(pillar_features, (0, 2, 1))
    feat_t = jnp.pad(feat_t, ((0, 0), (0, 0), (1, T - P - 1)))
    feat_flat = feat_t.reshape(B * C * T)

    out = _scatter_kernel(cells, feat_flat)
    return out.reshape(B, C, NX, NY)


# speedup vs baseline: 10.9869x; 1.4261x over previous
import functools

import jax
import jax.numpy as jnp
from jax import lax
from jax.experimental import pallas as pl
from jax.experimental.pallas import tpu as pltpu
from jax.experimental.pallas import tpu_sc as plsc

NX = 496
NY = 496
C = 64
B = 4
P = 25000

L = 16
NCELL = NX * NY
SLICE = NCELL // 16
P_PAD = 25600
T = 25008
W = 3968
N_WIN = NCELL // W
SENT = 1 << 28

_mesh = plsc.VectorSubcoreMesh(core_axis_name="c", subcore_axis_name="s")


@functools.partial(
    pl.kernel,
    out_type=jax.ShapeDtypeStruct((B, C, NX, NY), jnp.float32),
    mesh=_mesh,
    compiler_params=pltpu.CompilerParams(needs_layout_passes=False),
    scratch_types=[
        pltpu.VMEM((SLICE,), jnp.int32),
        pltpu.VMEM((P_PAD,), jnp.int32),
        pltpu.VMEM((2 * T,), jnp.float32),
        pltpu.VMEM((2 * W,), jnp.int32),
        pltpu.VMEM((4, W // NY, NY), jnp.float32),
        pltpu.VMEM_SHARED((NCELL,), jnp.int32),
        pltpu.SemaphoreType.DMA,
        pltpu.SemaphoreType.DMA,
        pltpu.SemaphoreType.DMA,
        pltpu.SemaphoreType.DMA,
    ],
)
def _scatter_kernel(cells_hbm, feat_hbm, out_hbm,
                    map_v, cells_v, tab_v, idx_v, outw_v, map_sh,
                    sem_i0, sem_i1, sem_o0, sem_o1):
    sem_i = (sem_i0, sem_i1)
    sem_o = (sem_o0, sem_o1)
    core = lax.axis_index("c")
    sub = lax.axis_index("s")
    iota = jnp.arange(L, dtype=jnp.int32)
    zeros_i = jnp.zeros((L,), jnp.int32)

    base = sub * SLICE

    for lb in range(2):
        b = 2 * core + lb

        pltpu.sync_copy(cells_hbm.at[b], cells_v)

        def _zero(i, _):
            map_v[pl.ds(i * L, L)] = zeros_i
            return 0
        lax.fori_loop(0, SLICE // L, _zero, 0)

        def _scan(k, _):
            cell = cells_v[pl.ds(k * L, L)]
            rel = cell - base
            ok = (rel >= 0) & (rel < SLICE)
            loc = jnp.clip(rel, 0, SLICE - 1)
            vals = k * L + 1 + iota
            plsc.store_scatter(map_v, [loc], vals, mask=ok)
            for _fix in range(2):
                rb = plsc.load_gather(map_v, [loc], mask=ok)
                redo = ok & (rb < vals)
                plsc.store_scatter(map_v, [loc], vals, mask=redo)
            return 0
        lax.fori_loop(0, P_PAD // L, _scan, 0)

        pltpu.sync_copy(map_v, map_sh.at[pl.ds(base, SLICE)])
        plsc.subcore_barrier()

        c_base = 4 * sub
        for p in range(2):
            c0 = c_base + 2 * p
            pltpu.sync_copy(feat_hbm.at[pl.ds((b * C + c0) * T, 2 * T)], tab_v)

            def idx_copy(w, j):
                return pltpu.make_async_copy(
                    map_sh.at[pl.ds(w * W, W)],
                    idx_v.at[pl.ds(j * W, W)], sem_i[j])

            def out_copy(w, j):
                return pltpu.make_async_copy(
                    outw_v.at[pl.ds(2 * j, 2)],
                    out_hbm.at[b, pl.ds(c0, 2), pl.ds(w * (W // NY), W // NY)],
                    sem_o[j])

            def gather_win(j):
                r0 = 2 * j
                for rr in range(W // NY):

                    def _chunk(i, _):
                        o = rr * NY + i * L
                        idx = idx_v[pl.ds(j * W + o, L)]
                        outw_v[r0, rr, pl.ds(i * L, L)] = plsc.load_gather(
                            tab_v, [idx])
                        outw_v[r0 + 1, rr, pl.ds(i * L, L)] = plsc.load_gather(
                            tab_v, [idx + T])
                        return 0
                    lax.fori_loop(0, NY // L, _chunk, 0)

            idx_copy(0, 0).start()
            idx_copy(1, 1).start()
            for j in range(2):
                idx_copy(j, j).wait()
                gather_win(j)
                out_copy(j, j).start()
                idx_copy(j + 2, j).start()

            def _dbl(k, _):
                for j in range(2):
                    w = 2 * k + j
                    idx_copy(w, j).wait()
                    out_copy(w - 2, j).wait()
                    gather_win(j)
                    out_copy(w, j).start()
                    idx_copy(jnp.minimum(w + 2, N_WIN - 1), j).start()
                return 0
            lax.fori_loop(1, N_WIN // 2, _dbl, 0)

            for j in range(2):
                idx_copy(N_WIN - 2 + j, j).wait()
                out_copy(N_WIN - 2 + j, j).wait()

        plsc.subcore_barrier()


def kernel(pillar_features, coords):
    x = coords[:, :, 1]
    y = coords[:, :, 2]
    keep = (x + y) > 0
    cells = jnp.where(keep, x * NY + y, SENT).astype(jnp.int32)
    cells = jnp.pad(cells, ((0, 0), (0, P_PAD - P)), constant_values=SENT)

    feat_t = jnp.transpose(pillar_features, (0, 2, 1))
    feat_t = jnp.pad(feat_t, ((0, 0), (0, 0), (1, T - P - 1)))
    feat_flat = feat_t.reshape(B * C * T)

    return _scatter_kernel(cells, feat_flat)


# speedup vs baseline: 26.0287x; 2.3691x over previous
import functools

import jax
import jax.numpy as jnp
from jax import lax
from jax.experimental import pallas as pl
from jax.experimental.pallas import tpu as pltpu
from jax.experimental.pallas import tpu_sc as plsc

NX = 496
NY = 496
C = 64
B = 4
P = 25000

L = 16
NCELL = NX * NY
SLICE = NCELL // 16
P_PAD = 25600
T = 25008
W = 3968
N_WIN = NCELL // W
SENT = 1 << 28

_mesh = plsc.VectorSubcoreMesh(core_axis_name="c", subcore_axis_name="s")


@functools.partial(
    pl.kernel,
    out_type=jax.ShapeDtypeStruct((B, C, NX, NY), jnp.float32),
    mesh=_mesh,
    compiler_params=pltpu.CompilerParams(needs_layout_passes=False),
    scratch_types=[
        pltpu.VMEM((SLICE,), jnp.int32),
        pltpu.VMEM((P_PAD,), jnp.int32),
        pltpu.VMEM((2 * T,), jnp.float32),
        pltpu.VMEM((2 * W,), jnp.int32),
        pltpu.VMEM((4, W // NY, NY), jnp.float32),
        pltpu.VMEM_SHARED((NCELL,), jnp.int32),
        pltpu.SemaphoreType.DMA,
        pltpu.SemaphoreType.DMA,
        pltpu.SemaphoreType.DMA,
        pltpu.SemaphoreType.DMA,
    ],
)
def _scatter_kernel(cells_hbm, feat_hbm, out_hbm,
                    map_v, cells_v, tab_v, idx_v, outw_v, map_sh,
                    sem_i0, sem_i1, sem_o0, sem_o1):
    sem_i = (sem_i0, sem_i1)
    sem_o = (sem_o0, sem_o1)
    core = lax.axis_index("c")
    sub = lax.axis_index("s")
    iota = jnp.arange(L, dtype=jnp.int32)
    zeros_i = jnp.zeros((L,), jnp.int32)

    base = sub * SLICE

    for lb in range(2):
        b = 2 * core + lb

        pltpu.sync_copy(cells_hbm.at[b], cells_v)

        def _zero(i, _):
            map_v[pl.ds(i * L, L)] = zeros_i
            return 0
        lax.fori_loop(0, SLICE // L, _zero, 0)

        def _scan(k, _):
            cell = cells_v[pl.ds(k * L, L)]
            rel = cell - base
            ok = (rel >= 0) & (rel < SLICE)
            loc = jnp.clip(rel, 0, SLICE - 1)
            vals = k * L + 1 + iota
            plsc.store_scatter(map_v, [loc], vals, mask=ok)
            for _fix in range(2):
                rb = plsc.load_gather(map_v, [loc], mask=ok)
                redo = ok & (rb < vals)
                plsc.store_scatter(map_v, [loc], vals, mask=redo)
            return 0
        lax.fori_loop(0, P_PAD // L, _scan, 0)

        pltpu.sync_copy(map_v, map_sh.at[pl.ds(base, SLICE)])
        plsc.subcore_barrier()

        c_base = 4 * sub
        for p in range(2):
            c0 = c_base + 2 * p
            pltpu.sync_copy(feat_hbm.at[pl.ds((b * C + c0) * T, 2 * T)], tab_v)

            def idx_copy(w, j):
                return pltpu.make_async_copy(
                    map_sh.at[pl.ds(w * W, W)],
                    idx_v.at[pl.ds(j * W, W)], sem_i[j])

            def out_copy(w, j):
                return pltpu.make_async_copy(
                    outw_v.at[pl.ds(2 * j, 2)],
                    out_hbm.at[b, pl.ds(c0, 2), pl.ds(w * (W // NY), W // NY)],
                    sem_o[j])

            def gather_win(j):
                r0 = 2 * j

                def _row(rr, _):
                    @plsc.parallel_loop(0, NY // L, unroll=4)
                    def _chunk(i):
                        o = rr * NY + i * L
                        idx = idx_v[pl.ds(j * W + o, L)]
                        outw_v[r0, rr, pl.ds(i * L, L)] = plsc.load_gather(
                            tab_v, [idx])
                        outw_v[r0 + 1, rr, pl.ds(i * L, L)] = plsc.load_gather(
                            tab_v, [idx + T])
                    return 0
                lax.fori_loop(0, W // NY, _row, 0)

            idx_copy(0, 0).start()
            idx_copy(1, 1).start()
            for j in range(2):
                idx_copy(j, j).wait()
                gather_win(j)
                out_copy(j, j).start()
                idx_copy(j + 2, j).start()

            def _dbl(k, _):
                for j in range(2):
                    w = 2 * k + j
                    idx_copy(w, j).wait()
                    out_copy(w - 2, j).wait()
                    gather_win(j)
                    out_copy(w, j).start()
                    idx_copy(jnp.minimum(w + 2, N_WIN - 1), j).start()
                return 0
            lax.fori_loop(1, N_WIN // 2, _dbl, 0)

            for j in range(2):
                idx_copy(N_WIN - 2 + j, j).wait()
                out_copy(N_WIN - 2 + j, j).wait()

        plsc.subcore_barrier()


def kernel(pillar_features, coords):
    x = coords[:, :, 1]
    y = coords[:, :, 2]
    keep = (x + y) > 0
    cells = jnp.where(keep, x * NY + y, SENT).astype(jnp.int32)
    cells = jnp.pad(cells, ((0, 0), (0, P_PAD - P)), constant_values=SENT)

    feat_t = jnp.transpose(pillar_features, (0, 2, 1))
    feat_t = jnp.pad(feat_t, ((0, 0), (0, 0), (1, T - P - 1)))
    feat_flat = feat_t.reshape(B * C * T)

    return _scatter_kernel(cells, feat_flat)


# speedup vs baseline: 27.1728x; 1.0440x over previous
import functools

import jax
import jax.numpy as jnp
from jax import lax
from jax.experimental import pallas as pl
from jax.experimental.pallas import tpu as pltpu
from jax.experimental.pallas import tpu_sc as plsc

NX = 496
NY = 496
C = 64
B = 4
P = 25000

L = 16
NCELL = NX * NY
SLICE = NCELL // 16
P_PAD = 25600
T = 25008
W = 3968
N_WIN = NCELL // W
SENT = 1 << 28

_mesh = plsc.VectorSubcoreMesh(core_axis_name="c", subcore_axis_name="s")
_params = pltpu.CompilerParams(needs_layout_passes=False)


@functools.partial(
    pl.kernel,
    out_type=jax.ShapeDtypeStruct((B * NCELL,), jnp.int32),
    mesh=_mesh,
    compiler_params=_params,
    scratch_types=[
        pltpu.VMEM((SLICE,), jnp.int32),
        pltpu.VMEM((P_PAD,), jnp.int32),
    ],
)
def _winner_kernel(cells_hbm, map_hbm, map_v, cells_v):
    core = lax.axis_index("c")
    sub = lax.axis_index("s")
    iota = jnp.arange(L, dtype=jnp.int32)
    zeros_i = jnp.zeros((L,), jnp.int32)
    base = sub * SLICE

    for lb in range(2):
        b = 2 * core + lb
        pltpu.sync_copy(cells_hbm.at[b], cells_v)

        def _zero(i, _):
            map_v[pl.ds(i * L, L)] = zeros_i
            return 0
        lax.fori_loop(0, SLICE // L, _zero, 0)

        def _scan(k, _):
            cell = cells_v[pl.ds(k * L, L)]
            rel = cell - base
            ok = (rel >= 0) & (rel < SLICE)
            loc = jnp.clip(rel, 0, SLICE - 1)
            vals = k * L + 1 + iota
            plsc.store_scatter(map_v, [loc], vals, mask=ok)
            for _fix in range(2):
                rb = plsc.load_gather(map_v, [loc], mask=ok)
                redo = ok & (rb < vals)
                plsc.store_scatter(map_v, [loc], vals, mask=redo)
            return 0
        lax.fori_loop(0, P_PAD // L, _scan, 0)

        pltpu.sync_copy(map_v, map_hbm.at[pl.ds(b * NCELL + base, SLICE)])


@functools.partial(
    pl.kernel,
    out_type=jax.ShapeDtypeStruct((B, C, NX, NY), jnp.float32),
    mesh=_mesh,
    compiler_params=_params,
    scratch_types=[
        pltpu.VMEM((2 * T,), jnp.float32),
        pltpu.VMEM((2 * W,), jnp.int32),
        pltpu.VMEM((4, W // NY, NY), jnp.float32),
        pltpu.SemaphoreType.DMA,
        pltpu.SemaphoreType.DMA,
        pltpu.SemaphoreType.DMA,
        pltpu.SemaphoreType.DMA,
    ],
)
def _expand_kernel(map_hbm, feat_hbm, out_hbm,
                   tab_v, idx_v, outw_v, sem_i0, sem_i1, sem_o0, sem_o1):
    sem_i = (sem_i0, sem_i1)
    sem_o = (sem_o0, sem_o1)
    core = lax.axis_index("c")
    sub = lax.axis_index("s")
    c_base = 4 * sub

    for lb in range(2):
        b = 2 * core + lb
        for p in range(2):
            c0 = c_base + 2 * p
            pltpu.sync_copy(feat_hbm.at[pl.ds((b * C + c0) * T, 2 * T)], tab_v)

            def idx_copy(w, j):
                return pltpu.make_async_copy(
                    map_hbm.at[pl.ds(b * NCELL + w * W, W)],
                    idx_v.at[pl.ds(j * W, W)], sem_i[j])

            def out_copy(w, j):
                return pltpu.make_async_copy(
                    outw_v.at[pl.ds(2 * j, 2)],
                    out_hbm.at[b, pl.ds(c0, 2), pl.ds(w * (W // NY), W // NY)],
                    sem_o[j])

            def gather_win(j):
                r0 = 2 * j

                def _row(rr, _):
                    @plsc.parallel_loop(0, NY // L, unroll=4)
                    def _chunk(i):
                        idx = idx_v[pl.ds(j * W + rr * NY + i * L, L)]
                        outw_v[r0, rr, pl.ds(i * L, L)] = plsc.load_gather(
                            tab_v, [idx])
                        outw_v[r0 + 1, rr, pl.ds(i * L, L)] = plsc.load_gather(
                            tab_v, [idx + T])
                    return 0
                lax.fori_loop(0, W // NY, _row, 0)

            idx_copy(0, 0).start()
            idx_copy(1, 1).start()
            for j in range(2):
                idx_copy(j, j).wait()
                gather_win(j)
                out_copy(j, j).start()
                idx_copy(j + 2, j).start()

            def _dbl(k, _):
                for j in range(2):
                    w = 2 * k + j
                    idx_copy(w, j).wait()
                    out_copy(w - 2, j).wait()
                    gather_win(j)
                    out_copy(w, j).start()
                    idx_copy(jnp.minimum(w + 2, N_WIN - 1), j).start()
                return 0
            lax.fori_loop(1, N_WIN // 2, _dbl, 0)

            for j in range(2):
                idx_copy(N_WIN - 2 + j, j).wait()
                out_copy(N_WIN - 2 + j, j).wait()


def kernel(pillar_features, coords):
    x = coords[:, :, 1]
    y = coords[:, :, 2]
    keep = (x + y) > 0
    cells = jnp.where(keep, x * NY + y, SENT).astype(jnp.int32)
    cells = jnp.pad(cells, ((0, 0), (0, P_PAD - P)), constant_values=SENT)

    feat_t = jnp.transpose(pillar_features, (0, 2, 1))
    feat_t = jnp.pad(feat_t, ((0, 0), (0, 0), (1, T - P - 1)))
    feat_flat = feat_t.reshape(B * C * T)

    winner_map = _winner_kernel(cells)
    return _expand_kernel(winner_map, feat_flat)


# speedup vs baseline: 27.2642x; 1.0034x over previous
import functools

import jax
import jax.numpy as jnp
from jax import lax
from jax.experimental import pallas as pl
from jax.experimental.pallas import tpu as pltpu
from jax.experimental.pallas import tpu_sc as plsc

NX = 496
NY = 496
C = 64
B = 4
P = 25000

L = 16
NCELL = NX * NY
SLICE = NCELL // 16
P_PAD = 25600
T = 25008
W = 3968
N_WIN = NCELL // W
SENT = 1 << 28

_mesh = plsc.VectorSubcoreMesh(core_axis_name="c", subcore_axis_name="s")
_params = pltpu.CompilerParams(needs_layout_passes=False)


@functools.partial(
    pl.kernel,
    out_type=jax.ShapeDtypeStruct((B * NCELL,), jnp.int32),
    mesh=_mesh,
    compiler_params=_params,
    scratch_types=[
        pltpu.VMEM((SLICE,), jnp.int32),
        pltpu.VMEM((P_PAD,), jnp.int32),
    ],
)
def _winner_kernel(cells_hbm, map_hbm, map_v, cells_v):
    core = lax.axis_index("c")
    sub = lax.axis_index("s")
    iota = jnp.arange(L, dtype=jnp.int32)
    zeros_i = jnp.zeros((L,), jnp.int32)
    base = sub * SLICE

    for lb in range(2):
        b = 2 * core + lb
        pltpu.sync_copy(cells_hbm.at[b], cells_v)

        def _zero(i, _):
            map_v[pl.ds(i * L, L)] = zeros_i
            return 0
        lax.fori_loop(0, SLICE // L, _zero, 0)

        def _scan(k2, _):
            locs, oks, valss = [], [], []
            for u in range(2):
                k = 2 * k2 + u
                cell = cells_v[pl.ds(k * L, L)]
                rel = cell - base
                ok = (rel >= 0) & (rel < SLICE)
                loc = jnp.clip(rel, 0, SLICE - 1)
                vals = k * L + 1 + iota
                plsc.store_scatter(map_v, [loc], vals, mask=ok)
                locs.append(loc)
                oks.append(ok)
                valss.append(vals)
            for _fix in range(2):
                for u in range(2):
                    rb = plsc.load_gather(map_v, [locs[u]], mask=oks[u])
                    redo = oks[u] & (rb < valss[u])
                    plsc.store_scatter(map_v, [locs[u]], valss[u], mask=redo)
            return 0
        lax.fori_loop(0, P_PAD // (2 * L), _scan, 0)

        pltpu.sync_copy(map_v, map_hbm.at[pl.ds(b * NCELL + base, SLICE)])


@functools.partial(
    pl.kernel,
    out_type=jax.ShapeDtypeStruct((B, C, NX, NY), jnp.float32),
    mesh=_mesh,
    compiler_params=_params,
    scratch_types=[
        pltpu.VMEM((2 * T,), jnp.float32),
        pltpu.VMEM((2 * W,), jnp.int32),
        pltpu.VMEM((4, W // NY, NY), jnp.float32),
        pltpu.SemaphoreType.DMA,
        pltpu.SemaphoreType.DMA,
        pltpu.SemaphoreType.DMA,
        pltpu.SemaphoreType.DMA,
    ],
)
def _expand_kernel(map_hbm, feat_hbm, out_hbm,
                   tab_v, idx_v, outw_v, sem_i0, sem_i1, sem_o0, sem_o1):
    sem_i = (sem_i0, sem_i1)
    sem_o = (sem_o0, sem_o1)
    core = lax.axis_index("c")
    sub = lax.axis_index("s")
    c_base = 4 * sub

    for lb in range(2):
        b = 2 * core + lb
        for p in range(2):
            c0 = c_base + 2 * p
            pltpu.sync_copy(feat_hbm.at[pl.ds((b * C + c0) * T, 2 * T)], tab_v)

            def idx_copy(w, j):
                return pltpu.make_async_copy(
                    map_hbm.at[pl.ds(b * NCELL + w * W, W)],
                    idx_v.at[pl.ds(j * W, W)], sem_i[j])

            def out_copy(w, j):
                return pltpu.make_async_copy(
                    outw_v.at[pl.ds(2 * j, 2)],
                    out_hbm.at[b, pl.ds(c0, 2), pl.ds(w * (W // NY), W // NY)],
                    sem_o[j])

            def gather_win(j):
                r0 = 2 * j

                def _row(rr, _):
                    @plsc.parallel_loop(0, NY // L, unroll=4)
                    def _chunk(i):
                        idx = idx_v[pl.ds(j * W + rr * NY + i * L, L)]
                        outw_v[r0, rr, pl.ds(i * L, L)] = plsc.load_gather(
                            tab_v, [idx])
                        outw_v[r0 + 1, rr, pl.ds(i * L, L)] = plsc.load_gather(
                            tab_v, [idx + T])
                    return 0
                lax.fori_loop(0, W // NY, _row, 0)

            idx_copy(0, 0).start()
            idx_copy(1, 1).start()
            for j in range(2):
                idx_copy(j, j).wait()
                gather_win(j)
                out_copy(j, j).start()
                idx_copy(j + 2, j).start()

            def _dbl(k, _):
                for j in range(2):
                    w = 2 * k + j
                    idx_copy(w, j).wait()
                    out_copy(w - 2, j).wait()
                    gather_win(j)
                    out_copy(w, j).start()
                    idx_copy(jnp.minimum(w + 2, N_WIN - 1), j).start()
                return 0
            lax.fori_loop(1, N_WIN // 2, _dbl, 0)

            for j in range(2):
                idx_copy(N_WIN - 2 + j, j).wait()
                out_copy(N_WIN - 2 + j, j).wait()


def kernel(pillar_features, coords):
    x = coords[:, :, 1]
    y = coords[:, :, 2]
    keep = (x + y) > 0
    cells = jnp.where(keep, x * NY + y, SENT).astype(jnp.int32)
    cells = jnp.pad(cells, ((0, 0), (0, P_PAD - P)), constant_values=SENT)

    feat_t = jnp.transpose(pillar_features, (0, 2, 1))
    feat_t = jnp.pad(feat_t, ((0, 0), (0, 0), (1, T - P - 1)))
    feat_flat = feat_t.reshape(B * C * T)

    winner_map = _winner_kernel(cells)
    return _expand_kernel(winner_map, feat_flat)


# speedup vs baseline: 30.0833x; 1.1034x over previous
import functools

import jax
import jax.numpy as jnp
from jax import lax
from jax.experimental import pallas as pl
from jax.experimental.pallas import tpu as pltpu
from jax.experimental.pallas import tpu_sc as plsc

NX = 496
NY = 496
C = 64
B = 4
P = 25000

L = 16
NCELL = NX * NY
SLICE = NCELL // 16
P_PAD = 25600
T = 25008
W = 1984
N_WIN = NCELL // W
SENT = 1 << 28

_mesh = plsc.VectorSubcoreMesh(core_axis_name="c", subcore_axis_name="s")
_params = pltpu.CompilerParams(needs_layout_passes=False)


@functools.partial(
    pl.kernel,
    out_type=jax.ShapeDtypeStruct((B * NCELL,), jnp.int32),
    mesh=_mesh,
    compiler_params=_params,
    scratch_types=[
        pltpu.VMEM((SLICE,), jnp.int32),
        pltpu.VMEM((P_PAD,), jnp.int32),
    ],
)
def _winner_kernel(cells_hbm, map_hbm, map_v, cells_v):
    core = lax.axis_index("c")
    sub = lax.axis_index("s")
    iota = jnp.arange(L, dtype=jnp.int32)
    zeros_i = jnp.zeros((L,), jnp.int32)
    base = sub * SLICE

    for lb in range(2):
        b = 2 * core + lb
        pltpu.sync_copy(cells_hbm.at[b], cells_v)

        def _zero(i, _):
            map_v[pl.ds(i * L, L)] = zeros_i
            return 0
        lax.fori_loop(0, SLICE // L, _zero, 0)

        def _scan(k2, _):
            locs, oks, valss = [], [], []
            for u in range(2):
                k = 2 * k2 + u
                cell = cells_v[pl.ds(k * L, L)]
                rel = cell - base
                ok = (rel >= 0) & (rel < SLICE)
                loc = jnp.clip(rel, 0, SLICE - 1)
                vals = k * L + 1 + iota
                plsc.store_scatter(map_v, [loc], vals, mask=ok)
                locs.append(loc)
                oks.append(ok)
                valss.append(vals)
            for _fix in range(2):
                for u in range(2):
                    rb = plsc.load_gather(map_v, [locs[u]], mask=oks[u])
                    redo = oks[u] & (rb < valss[u])
                    plsc.store_scatter(map_v, [locs[u]], valss[u], mask=redo)
            return 0
        lax.fori_loop(0, P_PAD // (2 * L), _scan, 0)

        pltpu.sync_copy(map_v, map_hbm.at[pl.ds(b * NCELL + base, SLICE)])


@functools.partial(
    pl.kernel,
    out_type=jax.ShapeDtypeStruct((B, C, NX, NY), jnp.float32),
    mesh=_mesh,
    compiler_params=_params,
    scratch_types=[
        pltpu.VMEM((4 * T,), jnp.float32),
        pltpu.VMEM((2 * W,), jnp.int32),
        pltpu.VMEM((8, W // NY, NY), jnp.float32),
        pltpu.SemaphoreType.DMA,
        pltpu.SemaphoreType.DMA,
        pltpu.SemaphoreType.DMA,
        pltpu.SemaphoreType.DMA,
    ],
)
def _expand_kernel(map_hbm, feat_hbm, out_hbm,
                   tab_v, idx_v, outw_v, sem_i0, sem_i1, sem_o0, sem_o1):
    sem_i = (sem_i0, sem_i1)
    sem_o = (sem_o0, sem_o1)
    core = lax.axis_index("c")
    sub = lax.axis_index("s")
    c_base = 4 * sub

    for lb in range(2):
        b = 2 * core + lb
        c0 = c_base
        pltpu.sync_copy(feat_hbm.at[pl.ds((b * C + c0) * T, 4 * T)], tab_v)

        def idx_copy(w, j):
            return pltpu.make_async_copy(
                map_hbm.at[pl.ds(b * NCELL + w * W, W)],
                idx_v.at[pl.ds(j * W, W)], sem_i[j])

        def out_copy(w, j):
            return pltpu.make_async_copy(
                outw_v.at[pl.ds(4 * j, 4)],
                out_hbm.at[b, pl.ds(c0, 4), pl.ds(w * (W // NY), W // NY)],
                sem_o[j])

        def gather_win(j):
            r0 = 4 * j

            def _row(rr, _):
                @plsc.parallel_loop(0, NY // L, unroll=4)
                def _chunk(i):
                    idx = idx_v[pl.ds(j * W + rr * NY + i * L, L)]
                    outw_v[r0, rr, pl.ds(i * L, L)] = plsc.load_gather(
                        tab_v, [idx])
                    outw_v[r0 + 1, rr, pl.ds(i * L, L)] = plsc.load_gather(
                        tab_v, [idx + T])
                    outw_v[r0 + 2, rr, pl.ds(i * L, L)] = plsc.load_gather(
                        tab_v, [idx + 2 * T])
                    outw_v[r0 + 3, rr, pl.ds(i * L, L)] = plsc.load_gather(
                        tab_v, [idx + 3 * T])
                return 0
            lax.fori_loop(0, W // NY, _row, 0)

        idx_copy(0, 0).start()
        idx_copy(1, 1).start()
        for j in range(2):
            idx_copy(j, j).wait()
            gather_win(j)
            out_copy(j, j).start()
            idx_copy(j + 2, j).start()

        def _dbl(k, _):
            for j in range(2):
                w = 2 * k + j
                idx_copy(w, j).wait()
                out_copy(w - 2, j).wait()
                gather_win(j)
                out_copy(w, j).start()
                idx_copy(jnp.minimum(w + 2, N_WIN - 1), j).start()
            return 0
        lax.fori_loop(1, N_WIN // 2, _dbl, 0)

        for j in range(2):
            idx_copy(N_WIN - 2 + j, j).wait()
            out_copy(N_WIN - 2 + j, j).wait()


def kernel(pillar_features, coords):
    x = coords[:, :, 1]
    y = coords[:, :, 2]
    keep = (x + y) > 0
    cells = jnp.where(keep, x * NY + y, SENT).astype(jnp.int32)
    cells = jnp.pad(cells, ((0, 0), (0, P_PAD - P)), constant_values=SENT)

    feat_t = jnp.transpose(pillar_features, (0, 2, 1))
    feat_t = jnp.pad(feat_t, ((0, 0), (0, 0), (1, T - P - 1)))
    feat_flat = feat_t.reshape(B * C * T)

    winner_map = _winner_kernel(cells)
    return _expand_kernel(winner_map, feat_flat)


# speedup vs baseline: 33.3755x; 1.1094x over previous
import functools

import jax
import jax.numpy as jnp
from jax import lax
from jax.experimental import pallas as pl
from jax.experimental.pallas import tpu as pltpu
from jax.experimental.pallas import tpu_sc as plsc

NX = 496
NY = 496
C = 64
B = 4
P = 25000

L = 16
NCELL = NX * NY
SLICE = NCELL // 16
P_PAD = 25600
T = P
SLICE1 = NCELL // 8
W = 1984
N_WIN = NCELL // W
SENT = 1 << 28

_mesh = plsc.VectorSubcoreMesh(core_axis_name="c", subcore_axis_name="s")
_params = pltpu.CompilerParams(needs_layout_passes=False)


@functools.partial(
    pl.kernel,
    out_type=jax.ShapeDtypeStruct((B * NCELL,), jnp.int32),
    mesh=_mesh,
    compiler_params=_params,
    scratch_types=[
        pltpu.VMEM((SLICE1,), jnp.int32),
        pltpu.VMEM((P_PAD,), jnp.int32),
    ],
)
def _winner_kernel(cells_hbm, map_hbm, map_v, cells_v):
    core = lax.axis_index("c")
    sub = lax.axis_index("s")
    iota = jnp.arange(L, dtype=jnp.int32)
    zeros_i = jnp.zeros((L,), jnp.int32)
    base = jnp.remainder(sub, 8) * SLICE1

    if True:
        b = 2 * core + sub // 8
        pltpu.sync_copy(cells_hbm.at[b], cells_v)

        def _zero(i, _):
            map_v[pl.ds(i * L, L)] = zeros_i
            return 0
        lax.fori_loop(0, SLICE1 // L, _zero, 0)

        def _scan(k2, _):
            locs, oks, valss = [], [], []
            for u in range(2):
                k = 2 * k2 + u
                cell = cells_v[pl.ds(k * L, L)]
                rel = cell - base
                ok = (rel >= 0) & (rel < SLICE1)
                loc = jnp.clip(rel, 0, SLICE1 - 1)
                vals = k * L + 1 + iota
                plsc.store_scatter(map_v, [loc], vals, mask=ok)
                locs.append(loc)
                oks.append(ok)
                valss.append(vals)
            for _fix in range(2):
                for u in range(2):
                    rb = plsc.load_gather(map_v, [locs[u]], mask=oks[u])
                    redo = oks[u] & (rb < valss[u])
                    plsc.store_scatter(map_v, [locs[u]], valss[u], mask=redo)
            return 0
        lax.fori_loop(0, P_PAD // (2 * L), _scan, 0)

        pltpu.sync_copy(map_v, map_hbm.at[pl.ds(b * NCELL + base, SLICE1)])


@functools.partial(
    pl.kernel,
    out_type=jax.ShapeDtypeStruct((B, C, NX, NY), jnp.float32),
    mesh=_mesh,
    compiler_params=_params,
    scratch_types=[
        pltpu.VMEM((4 * T,), jnp.float32),
        pltpu.VMEM((2 * W,), jnp.int32),
        pltpu.VMEM((8, W // NY, NY), jnp.float32),
        pltpu.SemaphoreType.DMA,
        pltpu.SemaphoreType.DMA,
        pltpu.SemaphoreType.DMA,
        pltpu.SemaphoreType.DMA,
    ],
)
def _expand_kernel(map_hbm, feat_hbm, out_hbm,
                   tab_v, idx_v, outw_v, sem_i0, sem_i1, sem_o0, sem_o1):
    sem_i = (sem_i0, sem_i1)
    sem_o = (sem_o0, sem_o1)
    core = lax.axis_index("c")
    sub = lax.axis_index("s")
    c_base = 4 * sub

    for lb in range(2):
        b = 2 * core + lb
        c0 = c_base
        pltpu.sync_copy(feat_hbm.at[pl.ds((b * C + c0) * T, 4 * T)], tab_v)

        def idx_copy(w, j):
            return pltpu.make_async_copy(
                map_hbm.at[pl.ds(b * NCELL + w * W, W)],
                idx_v.at[pl.ds(j * W, W)], sem_i[j])

        def out_copy(w, j):
            return pltpu.make_async_copy(
                outw_v.at[pl.ds(4 * j, 4)],
                out_hbm.at[b, pl.ds(c0, 4), pl.ds(w * (W // NY), W // NY)],
                sem_o[j])

        def gather_win(j):
            r0 = 4 * j

            def _row(rr, _):
                @plsc.parallel_loop(0, NY // L, unroll=4)
                def _chunk(i):
                    idx = idx_v[pl.ds(j * W + rr * NY + i * L, L)]
                    live = idx > 0
                    p0 = jnp.maximum(idx - 1, 0)
                    zero = jnp.zeros((L,), jnp.float32)
                    for ch in range(4):
                        g = plsc.load_gather(tab_v, [p0 + ch * T])
                        outw_v[r0 + ch, rr, pl.ds(i * L, L)] = jnp.where(
                            live, g, zero)
                return 0
            lax.fori_loop(0, W // NY, _row, 0)

        idx_copy(0, 0).start()
        idx_copy(1, 1).start()
        for j in range(2):
            idx_copy(j, j).wait()
            gather_win(j)
            out_copy(j, j).start()
            idx_copy(j + 2, j).start()

        def _dbl(k, _):
            for j in range(2):
                w = 2 * k + j
                idx_copy(w, j).wait()
                out_copy(w - 2, j).wait()
                gather_win(j)
                out_copy(w, j).start()
                idx_copy(jnp.minimum(w + 2, N_WIN - 1), j).start()
            return 0
        lax.fori_loop(1, N_WIN // 2, _dbl, 0)

        for j in range(2):
            idx_copy(N_WIN - 2 + j, j).wait()
            out_copy(N_WIN - 2 + j, j).wait()


def kernel(pillar_features, coords):
    x = coords[:, :, 1]
    y = coords[:, :, 2]
    keep = (x + y) > 0
    cells = jnp.where(keep, x * NY + y, SENT).astype(jnp.int32)
    cells = jnp.pad(cells, ((0, 0), (0, P_PAD - P)), constant_values=SENT)

    feat_t = jnp.transpose(pillar_features, (0, 2, 1))
    feat_flat = feat_t.reshape(B * C * T)

    winner_map = _winner_kernel(cells)
    return _expand_kernel(winner_map, feat_flat)
